# baseline (speedup 1.0000x reference)
import functools

import jax
import jax.numpy as jnp
from jax import lax
from jax.experimental import pallas as pl
from jax.experimental.pallas import tpu as pltpu
from jax.experimental.pallas import tpu_sc as plsc

N = 50000
E = 800000
G = 64
HID = 64
HALF = 25000
NS = 16
NC = 2
SEG = 1568
DEG_PAD = NS * SEG
DUMMY = HALF
ZROWS = HALF + 8
EPT = E // NS
CHUNK = 2000
NCHUNK = EPT // CHUNK
BATCH = 80
NBATCH = CHUNK // BATCH
ROWB = 128
NROWB = (N + ROWB - 1) // ROWB
BLK = 200
NBLK = N // BLK


def _sc_mesh():
    return plsc.VectorSubcoreMesh(
        core_axis_name="c", subcore_axis_name="s", num_cores=NC, num_subcores=NS
    )


_SC_PARAMS = pltpu.CompilerParams(use_tc_tiling_on_sc=False, needs_layout_passes=False)


def _k1_body(edge_src, edge_dst, x_names, x_types, name_table, type_table,
             cnt_out, nf_out, tf_out, slots_src, slots_dst, counts_out,
             src_st, dst_st, idxbuf, ones_v, seg_v, nidx_v, nrows_v, trows_v,
             comp_src, comp_dst, cnt16_v,
             deg_sh, gsem, ssem):
    c = lax.axis_index("c")
    s = lax.axis_index("s")
    w = c * NS + s
    lo = c * HALF

    for v in range(SEG // 16):
        seg_v[pl.ds(v * 16, 16)] = jnp.zeros((16,), jnp.float32)
    pltpu.sync_copy(seg_v, deg_sh.at[pl.ds(s * SEG, SEG)])
    for v in range(BATCH // 16):
        ones_v[pl.ds(v * 16, 16)] = jnp.ones((16,), jnp.float32)
    plsc.subcore_barrier()

    def chunk_body(ch, off):
        base = s * EPT + ch * CHUNK
        pltpu.sync_copy(edge_src.at[pl.ds(base, CHUNK)], src_st)
        pltpu.sync_copy(edge_dst.at[pl.ds(base, CHUNK)], dst_st)
        for v in range(CHUNK // 16):
            d16 = dst_st[pl.ds(v * 16, 16)]
            s16 = src_st[pl.ds(v * 16, 16)]
            inh = (d16 >= lo) & (d16 < lo + HALF)
            dloc = d16 - lo
            idx16 = jnp.where(inh, dloc, DUMMY)
            idxbuf[v // (BATCH // 16), pl.ds((v % (BATCH // 16)) * 16, 16)] = idx16
            plsc.store_compressed(comp_dst.at[pl.ds(off, 16)], dloc, mask=inh)
            plsc.store_compressed(comp_src.at[pl.ds(off, 16)], s16, mask=inh)
            off = off + plsc.all_reduce_population_count(inh)[0]
        descs = [
            pltpu.async_copy(ones_v, deg_sh.at[idxbuf.at[b]], ssem, add=True)
            for b in range(NBATCH)
        ]
        for d in descs:
            d.wait()
        return off

    off = lax.fori_loop(0, NCHUNK, chunk_body, jnp.int32(0))
    nch = (off + (CHUNK - 1)) // CHUNK
    pad_end = nch * CHUNK

    def pad_body(i, carry):
        @pl.when(off + i * 16 < pad_end)
        def _():
            comp_dst[pl.ds(off + i * 16, 16)] = jnp.full((16,), DUMMY, jnp.int32)
            comp_src[pl.ds(off + i * 16, 16)] = jnp.zeros((16,), jnp.int32)
        return carry

    lax.fori_loop(0, CHUNK // 16, pad_body, 0)
    cnt16_v[...] = jnp.full((16,), nch, jnp.int32)
    pltpu.sync_copy(cnt16_v, counts_out.at[w])
    pltpu.sync_copy(comp_src.at[pl.ds(0, EPT)], slots_src.at[w])
    pltpu.sync_copy(comp_dst.at[pl.ds(0, EPT)], slots_dst.at[w])

    plsc.subcore_barrier()
    start = jnp.minimum(s * SEG, HALF - SEG)
    pltpu.sync_copy(deg_sh.at[pl.ds(start, SEG)], seg_v)
    pltpu.sync_copy(seg_v, cnt_out.at[pl.ds(lo + start, SEG)])

    def gbatch(j, carry):
        b = w + NC * NS * j

        @pl.when(b < NROWB)
        def _():
            st = jnp.minimum(b * ROWB, N - ROWB)
            pltpu.sync_copy(x_names.at[pl.ds(st, ROWB)], nidx_v)
            pltpu.async_copy(name_table.at[nidx_v], nrows_v, gsem).wait()
            pltpu.sync_copy(nrows_v, nf_out.at[pl.ds(st, ROWB)])
            pltpu.sync_copy(x_types.at[pl.ds(st, ROWB)], nidx_v)
            pltpu.async_copy(type_table.at[nidx_v], trows_v, gsem).wait()
            pltpu.sync_copy(trows_v, tf_out.at[pl.ds(st, ROWB)])

        return carry

    lax.fori_loop(0, (NROWB + NC * NS - 1) // (NC * NS), gbatch, 0)


def _k1(edge_src, edge_dst, x_names, x_types, name_table, type_table):
    f = pl.kernel(
        _k1_body,
        out_type=[
            jax.ShapeDtypeStruct((N,), jnp.float32),
            jax.ShapeDtypeStruct((N, 64), jnp.float32),
            jax.ShapeDtypeStruct((N, 16), jnp.float32),
            jax.ShapeDtypeStruct((NC * NS, EPT), jnp.int32),
            jax.ShapeDtypeStruct((NC * NS, EPT), jnp.int32),
            jax.ShapeDtypeStruct((NC * NS, 16), jnp.int32),
        ],
        mesh=_sc_mesh(),
        scratch_types=[
            pltpu.VMEM((CHUNK,), jnp.int32),
            pltpu.VMEM((CHUNK,), jnp.int32),
            pltpu.VMEM((NBATCH, BATCH), jnp.int32),
            pltpu.VMEM((BATCH,), jnp.float32),
            pltpu.VMEM((SEG,), jnp.float32),
            pltpu.VMEM((ROWB,), jnp.int32),
            pltpu.VMEM((ROWB, 64), jnp.float32),
            pltpu.VMEM((ROWB, 16), jnp.float32),
            pltpu.VMEM((EPT + 2 * CHUNK,), jnp.int32),
            pltpu.VMEM((EPT + 2 * CHUNK,), jnp.int32),
            pltpu.VMEM((16,), jnp.int32),
            pltpu.VMEM_SHARED((DEG_PAD,), jnp.float32),
            pltpu.SemaphoreType.DMA,
            pltpu.SemaphoreType.DMA,
        ],
        compiler_params=_SC_PARAMS,
    )
    return f(edge_src, edge_dst, x_names, x_types, name_table, type_table)


SUBSEG = SEG // 8


def _k2_body(y, slots_src, slots_dst, counts, z_out,
             src_st, idx_st, cnt16_v, gbuf0, gbuf1, bounce, z_sh, gsem, ssem):
    c = lax.axis_index("c")
    s = lax.axis_index("s")
    w = c * NS + s
    lo = c * HALF
    start = jnp.minimum(s * SEG, HALF - SEG)
    gbufs = [gbuf0, gbuf1]

    pltpu.sync_copy(counts.at[w], cnt16_v)
    nch = cnt16_v[...][0]
    for k in range(8):
        pltpu.sync_copy(y.at[pl.ds(lo + start + k * SUBSEG, SUBSEG)], bounce)
        pltpu.sync_copy(bounce, z_sh.at[pl.ds(start + k * SUBSEG, SUBSEG)])
    plsc.subcore_barrier()

    def chunk_body(ch, carry):
        base = ch * CHUNK
        pltpu.sync_copy(slots_src.at[w, pl.ds(base, CHUNK)], src_st)
        pltpu.sync_copy(slots_dst.at[w, pl.ds(base, CHUNK)], idx_st)
        gd = [None] * NBATCH
        sd = [None] * NBATCH
        gd[0] = pltpu.async_copy(y.at[src_st.at[pl.ds(0, BATCH)]], gbufs[0], gsem)
        for b in range(NBATCH):
            if b >= 1:
                sd[b - 1].wait()
            if b + 1 < NBATCH:
                gd[b + 1] = pltpu.async_copy(
                    y.at[src_st.at[pl.ds((b + 1) * BATCH, BATCH)]],
                    gbufs[(b + 1) % 2], gsem)
            gd[b].wait()
            sd[b] = pltpu.async_copy(gbufs[b % 2],
                                     z_sh.at[idx_st.at[pl.ds(b * BATCH, BATCH)]],
                                     ssem, add=True)
        sd[NBATCH - 1].wait()
        return carry

    lax.fori_loop(0, nch, chunk_body, 0)
    plsc.subcore_barrier()
    for k in range(8):
        pltpu.sync_copy(z_sh.at[pl.ds(start + k * SUBSEG, SUBSEG)], bounce)
        pltpu.sync_copy(bounce, z_out.at[pl.ds(lo + start + k * SUBSEG, SUBSEG)])


def _k2(y, slots_src, slots_dst, counts):
    f = pl.kernel(
        _k2_body,
        out_type=jax.ShapeDtypeStruct((N, HID), jnp.float32),
        mesh=_sc_mesh(),
        scratch_types=[
            pltpu.VMEM((CHUNK,), jnp.int32),
            pltpu.VMEM((CHUNK,), jnp.int32),
            pltpu.VMEM((16,), jnp.int32),
            pltpu.VMEM((BATCH, HID), jnp.float32),
            pltpu.VMEM((BATCH, HID), jnp.float32),
            pltpu.VMEM((SUBSEG, HID), jnp.float32),
            pltpu.VMEM_SHARED((ZROWS, HID), jnp.float32),
            pltpu.SemaphoreType.DMA,
            pltpu.SemaphoreType.DMA,
        ],
        compiler_params=_SC_PARAMS,
    )
    return f(y, slots_src, slots_dst, counts)


def _tc1_body(nf, tf, bh, cnt, w1a, w1b, w1c, o):
    acc = jnp.dot(nf[...], w1a[...], preferred_element_type=jnp.float32)
    acc += jnp.dot(tf[...], w1b[...], preferred_element_type=jnp.float32)
    acc += jnp.dot(bh[...], w1c[...], preferred_element_type=jnp.float32)
    dinv = lax.rsqrt(cnt[...] + 1.0)
    o[...] = acc * dinv


def _tc1(nf, tf, bh, cnt2, w1a, w1b, w1c):
    return pl.pallas_call(
        _tc1_body,
        grid=(NBLK,),
        in_specs=[
            pl.BlockSpec((BLK, 64), lambda i: (i, 0)),
            pl.BlockSpec((BLK, 16), lambda i: (i, 0)),
            pl.BlockSpec((BLK, 48), lambda i: (i, 0)),
            pl.BlockSpec((BLK, 1), lambda i: (i, 0)),
            pl.BlockSpec((64, HID), lambda i: (0, 0)),
            pl.BlockSpec((16, HID), lambda i: (0, 0)),
            pl.BlockSpec((48, HID), lambda i: (0, 0)),
        ],
        out_specs=pl.BlockSpec((BLK, HID), lambda i: (i, 0)),
        out_shape=jax.ShapeDtypeStruct((N, HID), jnp.float32),
    )(nf, tf, bh, cnt2, w1a, w1b, w1c)


def _tc2_body(z, cnt, w2, b1r, o):
    dinv = lax.rsqrt(cnt[...] + 1.0)
    h = jnp.maximum(z[...] * dinv + b1r[...], 0.0)
    o[...] = jnp.dot(h, w2[...], preferred_element_type=jnp.float32) * dinv


def _tc2(z1, cnt2, W2, b1r):
    return pl.pallas_call(
        _tc2_body,
        grid=(NBLK,),
        in_specs=[
            pl.BlockSpec((BLK, HID), lambda i: (i, 0)),
            pl.BlockSpec((BLK, 1), lambda i: (i, 0)),
            pl.BlockSpec((HID, HID), lambda i: (0, 0)),
            pl.BlockSpec((1, HID), lambda i: (0, 0)),
        ],
        out_specs=pl.BlockSpec((BLK, HID), lambda i: (i, 0)),
        out_shape=jax.ShapeDtypeStruct((N, HID), jnp.float32),
    )(z1, cnt2, W2, b1r)


def _tc3_body(z, cnt, bat, b2r, wc, bcr, o, acc, gcnt):
    i = pl.program_id(0)

    @pl.when(i == 0)
    def _():
        acc[...] = jnp.zeros_like(acc)
        gcnt[...] = jnp.zeros_like(gcnt)

    dinv = lax.rsqrt(cnt[...] + 1.0)
    h = jnp.maximum(z[...] * dinv + b2r[...], 0.0)
    onehot = (bat[...] == lax.broadcasted_iota(jnp.int32, (BLK, G), 1)
              ).astype(jnp.float32)
    acc[...] += lax.dot_general(onehot, h, (((0,), (0,)), ((), ())),
                                preferred_element_type=jnp.float32)
    gcnt[...] += lax.dot_general(onehot, jnp.ones((BLK, 1), jnp.float32),
                                 (((0,), (0,)), ((), ())),
                                 preferred_element_type=jnp.float32)

    @pl.when(i == NBLK - 1)
    def _():
        pooled = acc[...] / jnp.maximum(gcnt[...], 1.0)
        o[...] = jnp.dot(pooled, wc[...], preferred_element_type=jnp.float32) \
            + bcr[...]


def _tc3(z2, cnt2, bat2, b2r, Wc, bcr):
    return pl.pallas_call(
        _tc3_body,
        grid=(NBLK,),
        in_specs=[
            pl.BlockSpec((BLK, HID), lambda i: (i, 0)),
            pl.BlockSpec((BLK, 1), lambda i: (i, 0)),
            pl.BlockSpec((BLK, 1), lambda i: (i, 0)),
            pl.BlockSpec((1, HID), lambda i: (0, 0)),
            pl.BlockSpec((HID, 2), lambda i: (0, 0)),
            pl.BlockSpec((1, 2), lambda i: (0, 0)),
        ],
        out_specs=pl.BlockSpec((G, 2), lambda i: (0, 0)),
        out_shape=jax.ShapeDtypeStruct((G, 2), jnp.float32),
        scratch_shapes=[
            pltpu.VMEM((G, HID), jnp.float32),
            pltpu.VMEM((G, 1), jnp.float32),
        ],
    )(z2, cnt2, bat2, b2r, Wc, bcr)


def kernel(x_names, x_types, x_behaviors, edge_index, batch,
           name_table, type_table, W1, b1, W2, b2, Wc, bc):
    xn = x_names.astype(jnp.int32)
    xt = x_types.astype(jnp.int32)
    bh = x_behaviors.astype(jnp.float32)
    esrc = edge_index[0].astype(jnp.int32)
    edst = edge_index[1].astype(jnp.int32)
    bat2 = batch.astype(jnp.int32).reshape(N, 1)
    nt = name_table.astype(jnp.float32)
    tt = type_table.astype(jnp.float32)
    cnt, nf, tf, slots_src, slots_dst, counts = _k1(esrc, edst, xn, xt, nt, tt)
    cnt2 = cnt.reshape(N, 1)

    W1f = W1.astype(jnp.float32)
    y1 = _tc1(nf, tf, bh, cnt2, W1f[:64], W1f[64:80], W1f[80:])
    z1 = _k2(y1, slots_src, slots_dst, counts)
    y2 = _tc2(z1, cnt2, W2.astype(jnp.float32),
              b1.astype(jnp.float32).reshape(1, HID))
    z2 = _k2(y2, slots_src, slots_dst, counts)
    return _tc3(z2, cnt2, bat2, b2.astype(jnp.float32).reshape(1, HID),
                Wc.astype(jnp.float32), bc.astype(jnp.float32).reshape(1, 2))


# speedup vs baseline: 1.2447x; 1.2447x over previous
import functools

import jax
import jax.numpy as jnp
from jax import lax
from jax.experimental import pallas as pl
from jax.experimental.pallas import tpu as pltpu
from jax.experimental.pallas import tpu_sc as plsc

N = 50000
E = 800000
G = 64
HID = 64
HALF = 25000
NS = 16
NC = 2
SEG = 1568
DEG_PAD = NS * SEG
DUMMY = HALF
ZROWS = HALF + 8
EPT = E // NS
CHUNK = 2000
NCHUNK = EPT // CHUNK
BATCH = 80
NBATCH = CHUNK // BATCH
ROWB = 128
NROWB = (N + ROWB - 1) // ROWB
BLK = 200
NBLK = N // BLK


def _sc_mesh():
    return plsc.VectorSubcoreMesh(
        core_axis_name="c", subcore_axis_name="s", num_cores=NC, num_subcores=NS
    )


_SC_PARAMS = pltpu.CompilerParams(use_tc_tiling_on_sc=False)


def _k1_body(edge_dst, x_names, x_types, name_table, type_table,
             cnt_out, nf_out, tf_out,
             dst_st, idxbuf, ones_v, seg_v, nidx_v, nidx_v2, tidx_v, tidx_v2,
             nrows_v, nrows_v2, trows_v, trows_v2,
             deg_sh, gsem, ssem):
    c = lax.axis_index("c")
    s = lax.axis_index("s")
    w = c * NS + s
    lo = c * HALF

    for v in range(SEG // 16):
        seg_v[pl.ds(v * 16, 16)] = jnp.zeros((16,), jnp.float32)
    pltpu.sync_copy(seg_v, deg_sh.at[pl.ds(s * SEG, SEG)])
    for v in range(BATCH // 16):
        ones_v[pl.ds(v * 16, 16)] = jnp.ones((16,), jnp.float32)
    plsc.subcore_barrier()

    def chunk_body(ch, carry):
        base = s * EPT + ch * CHUNK
        pltpu.sync_copy(edge_dst.at[pl.ds(base, CHUNK)], dst_st)
        for v in range(CHUNK // 16):
            d16 = dst_st[pl.ds(v * 16, 16)]
            inh = (d16 >= lo) & (d16 < lo + HALF)
            idx16 = jnp.where(inh, d16 - lo, DUMMY)
            idxbuf[v // (BATCH // 16), pl.ds((v % (BATCH // 16)) * 16, 16)] = idx16
        descs = [
            pltpu.async_copy(ones_v, deg_sh.at[idxbuf.at[b]], ssem, add=True)
            for b in range(NBATCH)
        ]
        for d in descs:
            d.wait()
        return carry

    lax.fori_loop(0, NCHUNK, chunk_body, 0)
    plsc.subcore_barrier()
    start = jnp.minimum(s * SEG, HALF - SEG)
    pltpu.sync_copy(deg_sh.at[pl.ds(start, SEG)], seg_v)
    pltpu.sync_copy(seg_v, cnt_out.at[pl.ds(lo + start, SEG)])

    nsteps = (NROWB + NC * NS - 1) // (NC * NS)
    nidx = [nidx_v, nidx_v2]
    tidx = [tidx_v, tidx_v2]
    nrows = [nrows_v, nrows_v2]
    trows = [trows_v, trows_v2]
    dn = [None] * nsteps
    dt = [None] * nsteps
    stv = [None] * nsteps
    for j in range(nsteps):
        st = jnp.minimum((w + NC * NS * j) * ROWB, N - ROWB)
        stv[j] = st
        pltpu.sync_copy(x_names.at[pl.ds(st, ROWB)], nidx[j % 2])
        pltpu.sync_copy(x_types.at[pl.ds(st, ROWB)], tidx[j % 2])
        dn[j] = pltpu.async_copy(name_table.at[nidx[j % 2]], nrows[j % 2], gsem)
        dt[j] = pltpu.async_copy(type_table.at[tidx[j % 2]], trows[j % 2], gsem)
        if j >= 1:
            dn[j - 1].wait()
            dt[j - 1].wait()
            pltpu.sync_copy(nrows[(j - 1) % 2], nf_out.at[pl.ds(stv[j - 1], ROWB)])
            pltpu.sync_copy(trows[(j - 1) % 2], tf_out.at[pl.ds(stv[j - 1], ROWB)])
    dn[nsteps - 1].wait()
    dt[nsteps - 1].wait()
    pltpu.sync_copy(nrows[(nsteps - 1) % 2],
                    nf_out.at[pl.ds(stv[nsteps - 1], ROWB)])
    pltpu.sync_copy(trows[(nsteps - 1) % 2],
                    tf_out.at[pl.ds(stv[nsteps - 1], ROWB)])


def _k1(edge_dst, x_names, x_types, name_table, type_table):
    f = pl.kernel(
        _k1_body,
        out_type=[
            jax.ShapeDtypeStruct((N,), jnp.float32),
            jax.ShapeDtypeStruct((N, 64), jnp.float32),
            jax.ShapeDtypeStruct((N, 16), jnp.float32),
        ],
        mesh=_sc_mesh(),
        scratch_types=[
            pltpu.VMEM((CHUNK,), jnp.int32),
            pltpu.VMEM((NBATCH, BATCH), jnp.int32),
            pltpu.VMEM((BATCH,), jnp.float32),
            pltpu.VMEM((SEG,), jnp.float32),
            pltpu.VMEM((ROWB,), jnp.int32),
            pltpu.VMEM((ROWB,), jnp.int32),
            pltpu.VMEM((ROWB,), jnp.int32),
            pltpu.VMEM((ROWB,), jnp.int32),
            pltpu.VMEM((ROWB, 64), jnp.float32),
            pltpu.VMEM((ROWB, 64), jnp.float32),
            pltpu.VMEM((ROWB, 16), jnp.float32),
            pltpu.VMEM((ROWB, 16), jnp.float32),
            pltpu.VMEM_SHARED((DEG_PAD,), jnp.float32),
            pltpu.SemaphoreType.DMA,
            pltpu.SemaphoreType.DMA,
        ],
        compiler_params=_SC_PARAMS,
    )
    return f(edge_dst, x_names, x_types, name_table, type_table)


SUBSEG = SEG // 16


def _k2_body(y, edge_src, edge_dst, z_out,
             src_st, dst_st, idxbuf, gbuf0, gbuf1, gbuf2, bounce,
             z_sh, gsem, ssem):
    c = lax.axis_index("c")
    s = lax.axis_index("s")
    lo = c * HALF
    start = jnp.minimum(s * SEG, HALF - SEG)
    gbufs = [gbuf0, gbuf1, gbuf2]

    for k in range(16):
        pltpu.sync_copy(y.at[pl.ds(lo + start + k * SUBSEG, SUBSEG)], bounce)
        pltpu.sync_copy(bounce, z_sh.at[pl.ds(start + k * SUBSEG, SUBSEG)])
    plsc.subcore_barrier()

    def chunk_body(ch, carry):
        base = s * EPT + ch * CHUNK
        pltpu.sync_copy(edge_src.at[pl.ds(base, CHUNK)], src_st)
        pltpu.sync_copy(edge_dst.at[pl.ds(base, CHUNK)], dst_st)
        for v in range(CHUNK // 16):
            d16 = dst_st[pl.ds(v * 16, 16)]
            inh = (d16 >= lo) & (d16 < lo + HALF)
            idx16 = jnp.where(inh, d16 - lo, DUMMY)
            idxbuf[v // (BATCH // 16), pl.ds((v % (BATCH // 16)) * 16, 16)] = idx16
        gd = [None] * NBATCH
        sd = [None] * NBATCH

        def gfire(b):
            return pltpu.async_copy(
                y.at[src_st.at[pl.ds(b * BATCH, BATCH)]], gbufs[b % 3], gsem)

        gd[0] = gfire(0)
        gd[1] = gfire(1)
        for b in range(NBATCH):
            if b >= 1:
                sd[b - 1].wait()
            if b + 2 < NBATCH:
                gd[b + 2] = gfire(b + 2)
            gd[b].wait()
            sd[b] = pltpu.async_copy(gbufs[b % 3], z_sh.at[idxbuf.at[b]], ssem,
                                     add=True)
        sd[NBATCH - 1].wait()
        return carry

    lax.fori_loop(0, NCHUNK, chunk_body, 0)
    plsc.subcore_barrier()
    for k in range(16):
        pltpu.sync_copy(z_sh.at[pl.ds(start + k * SUBSEG, SUBSEG)], bounce)
        pltpu.sync_copy(bounce, z_out.at[pl.ds(lo + start + k * SUBSEG, SUBSEG)])


def _k2(y, edge_src, edge_dst):
    f = pl.kernel(
        _k2_body,
        out_type=jax.ShapeDtypeStruct((N, HID), jnp.float32),
        mesh=_sc_mesh(),
        scratch_types=[
            pltpu.VMEM((CHUNK,), jnp.int32),
            pltpu.VMEM((CHUNK,), jnp.int32),
            pltpu.VMEM((NBATCH, BATCH), jnp.int32),
            pltpu.VMEM((BATCH, HID), jnp.float32),
            pltpu.VMEM((BATCH, HID), jnp.float32),
            pltpu.VMEM((BATCH, HID), jnp.float32),
            pltpu.VMEM((SUBSEG, HID), jnp.float32),
            pltpu.VMEM_SHARED((ZROWS, HID), jnp.float32),
            pltpu.SemaphoreType.DMA,
            pltpu.SemaphoreType.DMA,
        ],
        compiler_params=_SC_PARAMS,
    )
    return f(y, edge_src, edge_dst)


def _tc1_body(nf, tf, bh, cnt, w1a, w1b, w1c, o):
    acc = jnp.dot(nf[...], w1a[...], preferred_element_type=jnp.float32)
    acc += jnp.dot(tf[...], w1b[...], preferred_element_type=jnp.float32)
    acc += jnp.dot(bh[...], w1c[...], preferred_element_type=jnp.float32)
    dinv = lax.rsqrt(cnt[...] + 1.0)
    o[...] = acc * dinv


def _tc1(nf, tf, bh, cnt2, w1a, w1b, w1c):
    return pl.pallas_call(
        _tc1_body,
        grid=(NBLK,),
        in_specs=[
            pl.BlockSpec((BLK, 64), lambda i: (i, 0)),
            pl.BlockSpec((BLK, 16), lambda i: (i, 0)),
            pl.BlockSpec((BLK, 48), lambda i: (i, 0)),
            pl.BlockSpec((BLK, 1), lambda i: (i, 0)),
            pl.BlockSpec((64, HID), lambda i: (0, 0)),
            pl.BlockSpec((16, HID), lambda i: (0, 0)),
            pl.BlockSpec((48, HID), lambda i: (0, 0)),
        ],
        out_specs=pl.BlockSpec((BLK, HID), lambda i: (i, 0)),
        out_shape=jax.ShapeDtypeStruct((N, HID), jnp.float32),
    )(nf, tf, bh, cnt2, w1a, w1b, w1c)


def _tc2_body(z, cnt, w2, b1r, o):
    dinv = lax.rsqrt(cnt[...] + 1.0)
    h = jnp.maximum(z[...] * dinv + b1r[...], 0.0)
    o[...] = jnp.dot(h, w2[...], preferred_element_type=jnp.float32) * dinv


def _tc2(z1, cnt2, W2, b1r):
    return pl.pallas_call(
        _tc2_body,
        grid=(NBLK,),
        in_specs=[
            pl.BlockSpec((BLK, HID), lambda i: (i, 0)),
            pl.BlockSpec((BLK, 1), lambda i: (i, 0)),
            pl.BlockSpec((HID, HID), lambda i: (0, 0)),
            pl.BlockSpec((1, HID), lambda i: (0, 0)),
        ],
        out_specs=pl.BlockSpec((BLK, HID), lambda i: (i, 0)),
        out_shape=jax.ShapeDtypeStruct((N, HID), jnp.float32),
    )(z1, cnt2, W2, b1r)


def _tc3_body(z, cnt, bat, b2r, wc, bcr, o, acc, gcnt):
    i = pl.program_id(0)

    @pl.when(i == 0)
    def _():
        acc[...] = jnp.zeros_like(acc)
        gcnt[...] = jnp.zeros_like(gcnt)

    dinv = lax.rsqrt(cnt[...] + 1.0)
    h = jnp.maximum(z[...] * dinv + b2r[...], 0.0)
    onehot = (bat[...] == lax.broadcasted_iota(jnp.int32, (BLK, G), 1)
              ).astype(jnp.float32)
    acc[...] += lax.dot_general(onehot, h, (((0,), (0,)), ((), ())),
                                preferred_element_type=jnp.float32)
    gcnt[...] += lax.dot_general(onehot, jnp.ones((BLK, 1), jnp.float32),
                                 (((0,), (0,)), ((), ())),
                                 preferred_element_type=jnp.float32)

    @pl.when(i == NBLK - 1)
    def _():
        pooled = acc[...] / jnp.maximum(gcnt[...], 1.0)
        o[...] = jnp.dot(pooled, wc[...], preferred_element_type=jnp.float32) \
            + bcr[...]


def _tc3(z2, cnt2, bat2, b2r, Wc, bcr):
    return pl.pallas_call(
        _tc3_body,
        grid=(NBLK,),
        in_specs=[
            pl.BlockSpec((BLK, HID), lambda i: (i, 0)),
            pl.BlockSpec((BLK, 1), lambda i: (i, 0)),
            pl.BlockSpec((BLK, 1), lambda i: (i, 0)),
            pl.BlockSpec((1, HID), lambda i: (0, 0)),
            pl.BlockSpec((HID, 2), lambda i: (0, 0)),
            pl.BlockSpec((1, 2), lambda i: (0, 0)),
        ],
        out_specs=pl.BlockSpec((G, 2), lambda i: (0, 0)),
        out_shape=jax.ShapeDtypeStruct((G, 2), jnp.float32),
        scratch_shapes=[
            pltpu.VMEM((G, HID), jnp.float32),
            pltpu.VMEM((G, 1), jnp.float32),
        ],
    )(z2, cnt2, bat2, b2r, Wc, bcr)


def kernel(x_names, x_types, x_behaviors, edge_index, batch,
           name_table, type_table, W1, b1, W2, b2, Wc, bc):
    xn = x_names.astype(jnp.int32)
    xt = x_types.astype(jnp.int32)
    bh = x_behaviors.astype(jnp.float32)
    esrc = edge_index[0].astype(jnp.int32)
    edst = edge_index[1].astype(jnp.int32)
    bat2 = batch.astype(jnp.int32).reshape(N, 1)
    nt = name_table.astype(jnp.float32)
    tt = type_table.astype(jnp.float32)
    cnt, nf, tf = _k1(edst, xn, xt, nt, tt)
    cnt2 = cnt.reshape(N, 1)

    W1f = W1.astype(jnp.float32)
    y1 = _tc1(nf, tf, bh, cnt2, W1f[:64], W1f[64:80], W1f[80:])
    z1 = _k2(y1, esrc, edst)
    y2 = _tc2(z1, cnt2, W2.astype(jnp.float32),
              b1.astype(jnp.float32).reshape(1, HID))
    z2 = _k2(y2, esrc, edst)
    return _tc3(z2, cnt2, bat2, b2.astype(jnp.float32).reshape(1, HID),
                Wc.astype(jnp.float32), bc.astype(jnp.float32).reshape(1, 2))


# speedup vs baseline: 1.6051x; 1.2896x over previous
import functools

import jax
import jax.numpy as jnp
from jax import lax
from jax.experimental import pallas as pl
from jax.experimental.pallas import tpu as pltpu
from jax.experimental.pallas import tpu_sc as plsc

N = 50000
E = 800000
G = 64
HID = 64
HALF = 25000
NS = 16
NC = 2
SEG = 1568
DEG_PAD = NS * SEG
DUMMY = HALF
ZROWS = HALF + 8
EPT = E // NS
CHUNK = 2000
NCHUNK = EPT // CHUNK
BATCH = 80
NBATCH = CHUNK // BATCH
ROWB = 128
NROWB = (N + ROWB - 1) // ROWB
BLK = 200
NBLK = N // BLK


def _sc_mesh():
    return plsc.VectorSubcoreMesh(
        core_axis_name="c", subcore_axis_name="s", num_cores=NC, num_subcores=NS
    )


_SC_PARAMS = pltpu.CompilerParams(use_tc_tiling_on_sc=False)


def _k1_body(edge_dst, x_names, x_types, name_table, type_table,
             cnt_out, nf_out, tf_out,
             dst_st, idxbuf, ones_v, seg_v, nidx_v, nidx_v2, tidx_v, tidx_v2,
             nrows_v, nrows_v2, trows_v, trows_v2,
             deg_sh, gsem, ssem):
    c = lax.axis_index("c")
    s = lax.axis_index("s")
    w = c * NS + s
    lo = c * HALF

    for v in range(SEG // 16):
        seg_v[pl.ds(v * 16, 16)] = jnp.zeros((16,), jnp.float32)
    pltpu.sync_copy(seg_v, deg_sh.at[pl.ds(s * SEG, SEG)])
    for v in range(BATCH // 16):
        ones_v[pl.ds(v * 16, 16)] = jnp.ones((16,), jnp.float32)
    plsc.subcore_barrier()

    def chunk_body(ch, carry):
        base = s * EPT + ch * CHUNK
        pltpu.sync_copy(edge_dst.at[pl.ds(base, CHUNK)], dst_st)
        for v in range(CHUNK // 16):
            d16 = dst_st[pl.ds(v * 16, 16)]
            inh = (d16 >= lo) & (d16 < lo + HALF)
            idx16 = jnp.where(inh, d16 - lo, DUMMY)
            idxbuf[v // (BATCH // 16), pl.ds((v % (BATCH // 16)) * 16, 16)] = idx16
        descs = [
            pltpu.async_copy(ones_v, deg_sh.at[idxbuf.at[b]], ssem, add=True)
            for b in range(NBATCH)
        ]
        for d in descs:
            d.wait()
        return carry

    lax.fori_loop(0, NCHUNK, chunk_body, 0)
    plsc.subcore_barrier()
    start = jnp.minimum(s * SEG, HALF - SEG)
    pltpu.sync_copy(deg_sh.at[pl.ds(start, SEG)], seg_v)
    pltpu.sync_copy(seg_v, cnt_out.at[pl.ds(lo + start, SEG)])

    nsteps = (NROWB + NC * NS - 1) // (NC * NS)
    nidx = [nidx_v, nidx_v2]
    tidx = [tidx_v, tidx_v2]
    nrows = [nrows_v, nrows_v2]
    trows = [trows_v, trows_v2]
    dn = [None] * nsteps
    dt = [None] * nsteps
    stv = [None] * nsteps
    for j in range(nsteps):
        st = jnp.minimum((w + NC * NS * j) * ROWB, N - ROWB)
        stv[j] = st
        pltpu.sync_copy(x_names.at[pl.ds(st, ROWB)], nidx[j % 2])
        pltpu.sync_copy(x_types.at[pl.ds(st, ROWB)], tidx[j % 2])
        dn[j] = pltpu.async_copy(name_table.at[nidx[j % 2]], nrows[j % 2], gsem)
        dt[j] = pltpu.async_copy(type_table.at[tidx[j % 2]], trows[j % 2], gsem)
        if j >= 1:
            dn[j - 1].wait()
            dt[j - 1].wait()
            pltpu.sync_copy(nrows[(j - 1) % 2], nf_out.at[pl.ds(stv[j - 1], ROWB)])
            pltpu.sync_copy(trows[(j - 1) % 2], tf_out.at[pl.ds(stv[j - 1], ROWB)])
    dn[nsteps - 1].wait()
    dt[nsteps - 1].wait()
    pltpu.sync_copy(nrows[(nsteps - 1) % 2],
                    nf_out.at[pl.ds(stv[nsteps - 1], ROWB)])
    pltpu.sync_copy(trows[(nsteps - 1) % 2],
                    tf_out.at[pl.ds(stv[nsteps - 1], ROWB)])


def _k1(edge_dst, x_names, x_types, name_table, type_table):
    f = pl.kernel(
        _k1_body,
        out_type=[
            jax.ShapeDtypeStruct((N,), jnp.float32),
            jax.ShapeDtypeStruct((N, 64), jnp.float32),
            jax.ShapeDtypeStruct((N, 16), jnp.float32),
        ],
        mesh=_sc_mesh(),
        scratch_types=[
            pltpu.VMEM((CHUNK,), jnp.int32),
            pltpu.VMEM((NBATCH, BATCH), jnp.int32),
            pltpu.VMEM((BATCH,), jnp.float32),
            pltpu.VMEM((SEG,), jnp.float32),
            pltpu.VMEM((ROWB,), jnp.int32),
            pltpu.VMEM((ROWB,), jnp.int32),
            pltpu.VMEM((ROWB,), jnp.int32),
            pltpu.VMEM((ROWB,), jnp.int32),
            pltpu.VMEM((ROWB, 64), jnp.float32),
            pltpu.VMEM((ROWB, 64), jnp.float32),
            pltpu.VMEM((ROWB, 16), jnp.float32),
            pltpu.VMEM((ROWB, 16), jnp.float32),
            pltpu.VMEM_SHARED((DEG_PAD,), jnp.float32),
            pltpu.SemaphoreType.DMA,
            pltpu.SemaphoreType.DMA,
        ],
        compiler_params=_SC_PARAMS,
    )
    return f(edge_dst, x_names, x_types, name_table, type_table)


B2 = 40
C2 = 1000
NB2 = C2 // B2
NCH2 = E // (2 * NS) // C2
ZR2 = N + 48
SEG2 = ZR2 // NS
SUB2 = SEG2 // 8


def _k2_body(y, edge_src, edge_dst2d, zrows, z_out,
             src_st, idxbuf, gbuf0, gbuf1, gbuf2, gbuf3, bounce,
             z_sh, gsem, ssem):
    c = lax.axis_index("c")
    s = lax.axis_index("s")
    gbufs = [gbuf0, gbuf1, gbuf2, gbuf3]

    pltpu.sync_copy(zrows, bounce)
    for k in range(8):
        pltpu.sync_copy(bounce, z_sh.at[pl.ds(s * SEG2 + k * SUB2, SUB2)])
    plsc.subcore_barrier()

    ebase = c * (E // 2) + s * (E // (2 * NS))

    def chunk_body(ch, carry):
        base = ebase + ch * C2
        pltpu.sync_copy(edge_src.at[pl.ds(base, C2)], src_st)
        pltpu.sync_copy(edge_dst2d.at[pl.ds(base // B2, NB2)], idxbuf)
        gd = [None] * NB2
        sd = [None] * NB2

        def gfire(b):
            return pltpu.async_copy(
                y.at[src_st.at[pl.ds(b * B2, B2)]], gbufs[b % 4], gsem)

        gd[0] = gfire(0)
        gd[1] = gfire(1)
        gd[2] = gfire(2)
        for b in range(NB2):
            if b >= 1:
                sd[b - 1].wait()
            if b + 3 < NB2:
                gd[b + 3] = gfire(b + 3)
            gd[b].wait()
            sd[b] = pltpu.async_copy(gbufs[b % 4], z_sh.at[idxbuf.at[b]], ssem,
                                     add=True)
        sd[NB2 - 1].wait()
        return carry

    lax.fori_loop(0, NCH2, chunk_body, 0)
    plsc.subcore_barrier()
    start = jnp.minimum(s * SEG2, N - SEG2)
    for k in range(8):
        pltpu.sync_copy(z_sh.at[pl.ds(start + k * SUB2, SUB2)], bounce)
        pltpu.sync_copy(bounce, z_out.at[c, pl.ds(start + k * SUB2, SUB2)])


def _k2(y, edge_src, edge_dst2d, zrows):
    f = pl.kernel(
        _k2_body,
        out_type=jax.ShapeDtypeStruct((NC, N, HID), jnp.bfloat16),
        mesh=_sc_mesh(),
        scratch_types=[
            pltpu.VMEM((C2,), jnp.int32),
            pltpu.VMEM((NB2, B2), jnp.int32),
            pltpu.VMEM((B2, HID), jnp.bfloat16),
            pltpu.VMEM((B2, HID), jnp.bfloat16),
            pltpu.VMEM((B2, HID), jnp.bfloat16),
            pltpu.VMEM((B2, HID), jnp.bfloat16),
            pltpu.VMEM((SUB2, HID), jnp.bfloat16),
            pltpu.VMEM_SHARED((ZR2, HID), jnp.bfloat16),
            pltpu.SemaphoreType.DMA,
            pltpu.SemaphoreType.DMA,
        ],
        compiler_params=_SC_PARAMS,
    )
    return f(y, edge_src, edge_dst2d, zrows)


def _tc1_body(nf, tf, bh, cnt, w1a, w1b, w1c, o, ob):
    acc = jnp.dot(nf[...], w1a[...], preferred_element_type=jnp.float32)
    acc += jnp.dot(tf[...], w1b[...], preferred_element_type=jnp.float32)
    acc += jnp.dot(bh[...], w1c[...], preferred_element_type=jnp.float32)
    dinv = lax.rsqrt(cnt[...] + 1.0)
    y = acc * dinv
    o[...] = y
    ob[...] = y.astype(jnp.bfloat16)


def _tc1(nf, tf, bh, cnt2, w1a, w1b, w1c):
    return pl.pallas_call(
        _tc1_body,
        grid=(NBLK,),
        in_specs=[
            pl.BlockSpec((BLK, 64), lambda i: (i, 0)),
            pl.BlockSpec((BLK, 16), lambda i: (i, 0)),
            pl.BlockSpec((BLK, 48), lambda i: (i, 0)),
            pl.BlockSpec((BLK, 1), lambda i: (i, 0)),
            pl.BlockSpec((64, HID), lambda i: (0, 0)),
            pl.BlockSpec((16, HID), lambda i: (0, 0)),
            pl.BlockSpec((48, HID), lambda i: (0, 0)),
        ],
        out_specs=[pl.BlockSpec((BLK, HID), lambda i: (i, 0)),
                   pl.BlockSpec((BLK, HID), lambda i: (i, 0))],
        out_shape=[jax.ShapeDtypeStruct((N, HID), jnp.float32),
                   jax.ShapeDtypeStruct((N, HID), jnp.bfloat16)],
    )(nf, tf, bh, cnt2, w1a, w1b, w1c)


def _tc2_body(z0, z1, y, cnt, w2, b1r, o, ob):
    dinv = lax.rsqrt(cnt[...] + 1.0)
    zt = z0[...].astype(jnp.float32) + z1[...].astype(jnp.float32) + y[...]
    h = jnp.maximum(zt * dinv + b1r[...], 0.0)
    y2 = jnp.dot(h, w2[...], preferred_element_type=jnp.float32) * dinv
    o[...] = y2
    ob[...] = y2.astype(jnp.bfloat16)


def _tc2(z0, z1, y1, cnt2, W2, b1r):
    return pl.pallas_call(
        _tc2_body,
        grid=(NBLK,),
        in_specs=[
            pl.BlockSpec((BLK, HID), lambda i: (i, 0)),
            pl.BlockSpec((BLK, HID), lambda i: (i, 0)),
            pl.BlockSpec((BLK, HID), lambda i: (i, 0)),
            pl.BlockSpec((BLK, 1), lambda i: (i, 0)),
            pl.BlockSpec((HID, HID), lambda i: (0, 0)),
            pl.BlockSpec((1, HID), lambda i: (0, 0)),
        ],
        out_specs=[pl.BlockSpec((BLK, HID), lambda i: (i, 0)),
                   pl.BlockSpec((BLK, HID), lambda i: (i, 0))],
        out_shape=[jax.ShapeDtypeStruct((N, HID), jnp.float32),
                   jax.ShapeDtypeStruct((N, HID), jnp.bfloat16)],
    )(z0, z1, y1, cnt2, W2, b1r)


def _tc3_body(z0, z1, y, cnt, bat, b2r, wc, bcr, o, acc, gcnt):
    i = pl.program_id(0)

    @pl.when(i == 0)
    def _():
        acc[...] = jnp.zeros_like(acc)
        gcnt[...] = jnp.zeros_like(gcnt)

    dinv = lax.rsqrt(cnt[...] + 1.0)
    zt = z0[...].astype(jnp.float32) + z1[...].astype(jnp.float32) + y[...]
    h = jnp.maximum(zt * dinv + b2r[...], 0.0)
    onehot = (bat[...] == lax.broadcasted_iota(jnp.int32, (BLK, G), 1)
              ).astype(jnp.float32)
    acc[...] += lax.dot_general(onehot, h, (((0,), (0,)), ((), ())),
                                preferred_element_type=jnp.float32)
    gcnt[...] += lax.dot_general(onehot, jnp.ones((BLK, 1), jnp.float32),
                                 (((0,), (0,)), ((), ())),
                                 preferred_element_type=jnp.float32)

    @pl.when(i == NBLK - 1)
    def _():
        pooled = acc[...] / jnp.maximum(gcnt[...], 1.0)
        o[...] = jnp.dot(pooled, wc[...], preferred_element_type=jnp.float32) \
            + bcr[...]


def _tc3(z0, z1, y2, cnt2, bat2, b2r, Wc, bcr):
    return pl.pallas_call(
        _tc3_body,
        grid=(NBLK,),
        in_specs=[
            pl.BlockSpec((BLK, HID), lambda i: (i, 0)),
            pl.BlockSpec((BLK, HID), lambda i: (i, 0)),
            pl.BlockSpec((BLK, HID), lambda i: (i, 0)),
            pl.BlockSpec((BLK, 1), lambda i: (i, 0)),
            pl.BlockSpec((BLK, 1), lambda i: (i, 0)),
            pl.BlockSpec((1, HID), lambda i: (0, 0)),
            pl.BlockSpec((HID, 2), lambda i: (0, 0)),
            pl.BlockSpec((1, 2), lambda i: (0, 0)),
        ],
        out_specs=pl.BlockSpec((G, 2), lambda i: (0, 0)),
        out_shape=jax.ShapeDtypeStruct((G, 2), jnp.float32),
        scratch_shapes=[
            pltpu.VMEM((G, HID), jnp.float32),
            pltpu.VMEM((G, 1), jnp.float32),
        ],
    )(z0, z1, y2, cnt2, bat2, b2r, Wc, bcr)


def kernel(x_names, x_types, x_behaviors, edge_index, batch,
           name_table, type_table, W1, b1, W2, b2, Wc, bc):
    xn = x_names.astype(jnp.int32)
    xt = x_types.astype(jnp.int32)
    bh = x_behaviors.astype(jnp.float32)
    esrc = edge_index[0].astype(jnp.int32)
    edst = edge_index[1].astype(jnp.int32)
    bat2 = batch.astype(jnp.int32).reshape(N, 1)
    nt = name_table.astype(jnp.float32)
    tt = type_table.astype(jnp.float32)
    cnt, nf, tf = _k1(edst, xn, xt, nt, tt)
    cnt2 = cnt.reshape(N, 1)

    edst2d = edst.reshape(E // B2, B2)
    zrows = jnp.zeros((SUB2, HID), jnp.bfloat16)

    W1f = W1.astype(jnp.float32)
    y1, y1b = _tc1(nf, tf, bh, cnt2, W1f[:64], W1f[64:80], W1f[80:])
    z1 = _k2(y1b, esrc, edst2d, zrows)
    y2, y2b = _tc2(z1[0], z1[1], y1, cnt2, W2.astype(jnp.float32),
                   b1.astype(jnp.float32).reshape(1, HID))
    z2 = _k2(y2b, esrc, edst2d, zrows)
    return _tc3(z2[0], z2[1], y2, cnt2, bat2,
                b2.astype(jnp.float32).reshape(1, HID),
                Wc.astype(jnp.float32), bc.astype(jnp.float32).reshape(1, 2))


# speedup vs baseline: 2.0584x; 1.2824x over previous
import functools

import jax
import jax.numpy as jnp
from jax import lax
from jax.experimental import pallas as pl
from jax.experimental.pallas import tpu as pltpu
from jax.experimental.pallas import tpu_sc as plsc

N = 50000
E = 800000
G = 64
HID = 64
HALF = 25000
NS = 16
NC = 2
SEG = 1568
DEG_PAD = NS * SEG
DUMMY = HALF
ZROWS = HALF + 8
EPT = E // NS
CHUNK = 2000
NCHUNK = EPT // CHUNK
BATCH = 80
NBATCH = CHUNK // BATCH
ROWB = 128
NROWB = (N + ROWB - 1) // ROWB
BLK = 200
NBLK = N // BLK


def _sc_mesh():
    return plsc.VectorSubcoreMesh(
        core_axis_name="c", subcore_axis_name="s", num_cores=NC, num_subcores=NS
    )


_SC_PARAMS = pltpu.CompilerParams(use_tc_tiling_on_sc=False)


def _k1_body(edge_dst2d, x_names, x_types, name_table, type_table,
             cnt_out, nf_out, tf_out,
             idxbuf, ones_v, seg_v, nidx_v, nidx_v2, tidx_v, tidx_v2,
             nrows_v, nrows_v2, trows_v, trows_v2,
             deg_sh, gsem, ssem):
    c = lax.axis_index("c")
    s = lax.axis_index("s")
    w = c * NS + s

    for v in range(SEG2 // 16):
        seg_v[pl.ds(v * 16, 16)] = jnp.zeros((16,), jnp.float32)
    seg_v[pl.ds(SEG2 - 16, 16)] = jnp.zeros((16,), jnp.float32)
    pltpu.sync_copy(seg_v, deg_sh.at[pl.ds(s * SEG2, SEG2)])
    for off in (0, 16, B2 - 16):
        ones_v[pl.ds(off, 16)] = jnp.ones((16,), jnp.float32)
    plsc.subcore_barrier()

    ebase = c * (E // 2) + s * (E // (2 * NS))

    def chunk_body(ch, carry):
        base = ebase + ch * C2
        pltpu.sync_copy(edge_dst2d.at[pl.ds(base // B2, NB2)], idxbuf)
        descs = [
            pltpu.async_copy(ones_v, deg_sh.at[idxbuf.at[b]], ssem, add=True)
            for b in range(NB2)
        ]
        for d in descs:
            d.wait()
        return carry

    lax.fori_loop(0, NCH2, chunk_body, 0)
    plsc.subcore_barrier()
    start = jnp.minimum(s * SEG2, N - SEG2)
    pltpu.sync_copy(deg_sh.at[pl.ds(start, SEG2)], seg_v)
    pltpu.sync_copy(seg_v, cnt_out.at[c, pl.ds(start, SEG2)])

    nsteps = (NROWB + NC * NS - 1) // (NC * NS)
    nidx = [nidx_v, nidx_v2]
    tidx = [tidx_v, tidx_v2]
    nrows = [nrows_v, nrows_v2]
    trows = [trows_v, trows_v2]
    dn = [None] * nsteps
    dt = [None] * nsteps
    stv = [None] * nsteps
    for j in range(nsteps):
        st = jnp.minimum((w + NC * NS * j) * ROWB, N - ROWB)
        stv[j] = st
        pltpu.sync_copy(x_names.at[pl.ds(st, ROWB)], nidx[j % 2])
        pltpu.sync_copy(x_types.at[pl.ds(st, ROWB)], tidx[j % 2])
        dn[j] = pltpu.async_copy(name_table.at[nidx[j % 2]], nrows[j % 2], gsem)
        dt[j] = pltpu.async_copy(type_table.at[tidx[j % 2]], trows[j % 2], gsem)
        if j >= 1:
            dn[j - 1].wait()
            dt[j - 1].wait()
            pltpu.sync_copy(nrows[(j - 1) % 2], nf_out.at[pl.ds(stv[j - 1], ROWB)])
            pltpu.sync_copy(trows[(j - 1) % 2], tf_out.at[pl.ds(stv[j - 1], ROWB)])
    dn[nsteps - 1].wait()
    dt[nsteps - 1].wait()
    pltpu.sync_copy(nrows[(nsteps - 1) % 2],
                    nf_out.at[pl.ds(stv[nsteps - 1], ROWB)])
    pltpu.sync_copy(trows[(nsteps - 1) % 2],
                    tf_out.at[pl.ds(stv[nsteps - 1], ROWB)])


def _k1(edge_dst2d, x_names, x_types, name_table, type_table):
    f = pl.kernel(
        _k1_body,
        out_type=[
            jax.ShapeDtypeStruct((NC, N), jnp.float32),
            jax.ShapeDtypeStruct((N, 64), jnp.float32),
            jax.ShapeDtypeStruct((N, 16), jnp.float32),
        ],
        mesh=_sc_mesh(),
        scratch_types=[
            pltpu.VMEM((NB2, B2), jnp.int32),
            pltpu.VMEM((B2,), jnp.float32),
            pltpu.VMEM((SEG2,), jnp.float32),
            pltpu.VMEM((ROWB,), jnp.int32),
            pltpu.VMEM((ROWB,), jnp.int32),
            pltpu.VMEM((ROWB,), jnp.int32),
            pltpu.VMEM((ROWB,), jnp.int32),
            pltpu.VMEM((ROWB, 64), jnp.float32),
            pltpu.VMEM((ROWB, 64), jnp.float32),
            pltpu.VMEM((ROWB, 16), jnp.float32),
            pltpu.VMEM((ROWB, 16), jnp.float32),
            pltpu.VMEM_SHARED((ZR2,), jnp.float32),
            pltpu.SemaphoreType.DMA,
            pltpu.SemaphoreType.DMA,
        ],
        compiler_params=_SC_PARAMS,
    )
    return f(edge_dst2d, x_names, x_types, name_table, type_table)


B2 = 40
C2 = 1000
NB2 = C2 // B2
NCH2 = E // (2 * NS) // C2
ZR2 = N + 48
SEG2 = ZR2 // NS
SUB2 = SEG2 // 8


def _k2_body(y, edge_src, edge_dst2d, zrows, z_out,
             src_st, idxbuf, gbuf0, gbuf1, gbuf2, gbuf3, bounce,
             z_sh, gsem, ssem):
    c = lax.axis_index("c")
    s = lax.axis_index("s")
    gbufs = [gbuf0, gbuf1, gbuf2, gbuf3]

    pltpu.sync_copy(zrows, bounce)
    for k in range(8):
        pltpu.sync_copy(bounce, z_sh.at[pl.ds(s * SEG2 + k * SUB2, SUB2)])
    plsc.subcore_barrier()

    ebase = c * (E // 2) + s * (E // (2 * NS))

    def chunk_body(ch, carry):
        base = ebase + ch * C2
        pltpu.sync_copy(edge_src.at[pl.ds(base, C2)], src_st)
        pltpu.sync_copy(edge_dst2d.at[pl.ds(base // B2, NB2)], idxbuf)
        gd = [None] * NB2
        sd = [None] * NB2

        def gfire(b):
            return pltpu.async_copy(
                y.at[src_st.at[pl.ds(b * B2, B2)]], gbufs[b % 4], gsem)

        gd[0] = gfire(0)
        gd[1] = gfire(1)
        gd[2] = gfire(2)
        for b in range(NB2):
            if b >= 1:
                sd[b - 1].wait()
            if b + 3 < NB2:
                gd[b + 3] = gfire(b + 3)
            gd[b].wait()
            sd[b] = pltpu.async_copy(gbufs[b % 4], z_sh.at[idxbuf.at[b]], ssem,
                                     add=True)
        sd[NB2 - 1].wait()
        return carry

    lax.fori_loop(0, NCH2, chunk_body, 0)
    plsc.subcore_barrier()
    start = jnp.minimum(s * SEG2, N - SEG2)
    for k in range(8):
        pltpu.sync_copy(z_sh.at[pl.ds(start + k * SUB2, SUB2)], bounce)
        pltpu.sync_copy(bounce, z_out.at[c, pl.ds(start + k * SUB2, SUB2)])


def _k2(y, edge_src, edge_dst2d, zrows):
    f = pl.kernel(
        _k2_body,
        out_type=jax.ShapeDtypeStruct((NC, N, HID), jnp.bfloat16),
        mesh=_sc_mesh(),
        scratch_types=[
            pltpu.VMEM((C2,), jnp.int32),
            pltpu.VMEM((NB2, B2), jnp.int32),
            pltpu.VMEM((B2, HID), jnp.bfloat16),
            pltpu.VMEM((B2, HID), jnp.bfloat16),
            pltpu.VMEM((B2, HID), jnp.bfloat16),
            pltpu.VMEM((B2, HID), jnp.bfloat16),
            pltpu.VMEM((SUB2, HID), jnp.bfloat16),
            pltpu.VMEM_SHARED((ZR2, HID), jnp.bfloat16),
            pltpu.SemaphoreType.DMA,
            pltpu.SemaphoreType.DMA,
        ],
        compiler_params=_SC_PARAMS,
    )
    return f(y, edge_src, edge_dst2d, zrows)


def _tc1_body(nf, tf, bh, cnt0, cnt1, w1a, w1b, w1c, o, ob):
    acc = jnp.dot(nf[...], w1a[...], preferred_element_type=jnp.float32)
    acc += jnp.dot(tf[...], w1b[...], preferred_element_type=jnp.float32)
    acc += jnp.dot(bh[...], w1c[...], preferred_element_type=jnp.float32)
    dinv = lax.rsqrt(cnt0[...] + cnt1[...] + 1.0)
    y = acc * dinv
    o[...] = y
    ob[...] = y.astype(jnp.bfloat16)


def _tc1(nf, tf, bh, cnt0, cnt1, w1a, w1b, w1c):
    return pl.pallas_call(
        _tc1_body,
        grid=(NBLK,),
        in_specs=[
            pl.BlockSpec((BLK, 64), lambda i: (i, 0)),
            pl.BlockSpec((BLK, 16), lambda i: (i, 0)),
            pl.BlockSpec((BLK, 48), lambda i: (i, 0)),
            pl.BlockSpec((BLK, 1), lambda i: (i, 0)),
            pl.BlockSpec((BLK, 1), lambda i: (i, 0)),
            pl.BlockSpec((64, HID), lambda i: (0, 0)),
            pl.BlockSpec((16, HID), lambda i: (0, 0)),
            pl.BlockSpec((48, HID), lambda i: (0, 0)),
        ],
        out_specs=[pl.BlockSpec((BLK, HID), lambda i: (i, 0)),
                   pl.BlockSpec((BLK, HID), lambda i: (i, 0))],
        out_shape=[jax.ShapeDtypeStruct((N, HID), jnp.float32),
                   jax.ShapeDtypeStruct((N, HID), jnp.bfloat16)],
    )(nf, tf, bh, cnt0, cnt1, w1a, w1b, w1c)


def _tc2_body(z0, z1, y, cnt0, cnt1, w2, b1r, o, ob):
    dinv = lax.rsqrt(cnt0[...] + cnt1[...] + 1.0)
    zt = z0[...].astype(jnp.float32) + z1[...].astype(jnp.float32) + y[...]
    h = jnp.maximum(zt * dinv + b1r[...], 0.0)
    y2 = jnp.dot(h, w2[...], preferred_element_type=jnp.float32) * dinv
    o[...] = y2
    ob[...] = y2.astype(jnp.bfloat16)


def _tc2(z0, z1, y1, cnt0, cnt1, W2, b1r):
    return pl.pallas_call(
        _tc2_body,
        grid=(NBLK,),
        in_specs=[
            pl.BlockSpec((BLK, HID), lambda i: (i, 0)),
            pl.BlockSpec((BLK, HID), lambda i: (i, 0)),
            pl.BlockSpec((BLK, HID), lambda i: (i, 0)),
            pl.BlockSpec((BLK, 1), lambda i: (i, 0)),
            pl.BlockSpec((BLK, 1), lambda i: (i, 0)),
            pl.BlockSpec((HID, HID), lambda i: (0, 0)),
            pl.BlockSpec((1, HID), lambda i: (0, 0)),
        ],
        out_specs=[pl.BlockSpec((BLK, HID), lambda i: (i, 0)),
                   pl.BlockSpec((BLK, HID), lambda i: (i, 0))],
        out_shape=[jax.ShapeDtypeStruct((N, HID), jnp.float32),
                   jax.ShapeDtypeStruct((N, HID), jnp.bfloat16)],
    )(z0, z1, y1, cnt0, cnt1, W2, b1r)


def _tc3_body(z0, z1, y, cnt0, cnt1, bat, b2r, wc, bcr, o, acc, gcnt):
    i = pl.program_id(0)

    @pl.when(i == 0)
    def _():
        acc[...] = jnp.zeros_like(acc)
        gcnt[...] = jnp.zeros_like(gcnt)

    dinv = lax.rsqrt(cnt0[...] + cnt1[...] + 1.0)
    zt = z0[...].astype(jnp.float32) + z1[...].astype(jnp.float32) + y[...]
    h = jnp.maximum(zt * dinv + b2r[...], 0.0)
    onehot = (bat[...] == lax.broadcasted_iota(jnp.int32, (BLK, G), 1)
              ).astype(jnp.float32)
    acc[...] += lax.dot_general(onehot, h, (((0,), (0,)), ((), ())),
                                preferred_element_type=jnp.float32)
    gcnt[...] += lax.dot_general(onehot, jnp.ones((BLK, 1), jnp.float32),
                                 (((0,), (0,)), ((), ())),
                                 preferred_element_type=jnp.float32)

    @pl.when(i == NBLK - 1)
    def _():
        pooled = acc[...] / jnp.maximum(gcnt[...], 1.0)
        o[...] = jnp.dot(pooled, wc[...], preferred_element_type=jnp.float32) \
            + bcr[...]


def _tc3(z0, z1, y2, cnt0, cnt1, bat2, b2r, Wc, bcr):
    return pl.pallas_call(
        _tc3_body,
        grid=(NBLK,),
        in_specs=[
            pl.BlockSpec((BLK, HID), lambda i: (i, 0)),
            pl.BlockSpec((BLK, HID), lambda i: (i, 0)),
            pl.BlockSpec((BLK, HID), lambda i: (i, 0)),
            pl.BlockSpec((BLK, 1), lambda i: (i, 0)),
            pl.BlockSpec((BLK, 1), lambda i: (i, 0)),
            pl.BlockSpec((BLK, 1), lambda i: (i, 0)),
            pl.BlockSpec((1, HID), lambda i: (0, 0)),
            pl.BlockSpec((HID, 2), lambda i: (0, 0)),
            pl.BlockSpec((1, 2), lambda i: (0, 0)),
        ],
        out_specs=pl.BlockSpec((G, 2), lambda i: (0, 0)),
        out_shape=jax.ShapeDtypeStruct((G, 2), jnp.float32),
        scratch_shapes=[
            pltpu.VMEM((G, HID), jnp.float32),
            pltpu.VMEM((G, 1), jnp.float32),
        ],
    )(z0, z1, y2, cnt0, cnt1, bat2, b2r, Wc, bcr)


def kernel(x_names, x_types, x_behaviors, edge_index, batch,
           name_table, type_table, W1, b1, W2, b2, Wc, bc):
    xn = x_names.astype(jnp.int32)
    xt = x_types.astype(jnp.int32)
    bh = x_behaviors.astype(jnp.float32)
    esrc = edge_index[0].astype(jnp.int32)
    edst = edge_index[1].astype(jnp.int32)
    bat2 = batch.astype(jnp.int32).reshape(N, 1)
    nt = name_table.astype(jnp.float32)
    tt = type_table.astype(jnp.float32)
    edst2d0 = edst.reshape(E // B2, B2)
    cntp, nf, tf = _k1(edst2d0, xn, xt, nt, tt)
    cnt0 = cntp[0].reshape(N, 1)
    cnt1 = cntp[1].reshape(N, 1)

    zrows = jnp.zeros((SUB2, HID), jnp.bfloat16)

    W1f = W1.astype(jnp.float32)
    y1, y1b = _tc1(nf, tf, bh, cnt0, cnt1, W1f[:64], W1f[64:80], W1f[80:])
    z1 = _k2(y1b, esrc, edst2d0, zrows)
    y2, y2b = _tc2(z1[0], z1[1], y1, cnt0, cnt1, W2.astype(jnp.float32),
                   b1.astype(jnp.float32).reshape(1, HID))
    z2 = _k2(y2b, esrc, edst2d0, zrows)
    return _tc3(z2[0], z2[1], y2, cnt0, cnt1, bat2,
                b2.astype(jnp.float32).reshape(1, HID),
                Wc.astype(jnp.float32), bc.astype(jnp.float32).reshape(1, 2))
